# Initial kernel scaffold; baseline (speedup 1.0000x reference)
import jax
import jax.numpy as jnp
from jax.experimental import pallas as pl


def kernel(gradient):
    raise NotImplementedError("write your pallas kernel here")


# speedup vs baseline: 71.9079x; 71.9079x over previous
import functools

import jax
import jax.numpy as jnp
import numpy as np
from jax.experimental import pallas as pl

_ROWS = 2048
_DIM = 4096
_HALF = _DIM // 2
_BW = 256
_CAP = _BW - 1

_BM = 256
_BNF = 256
_BNI = 512

_PREC = jax.lax.Precision.HIGHEST


def _trig_tables():
    n = np.arange(_DIM)
    k = np.arange(_HALF)
    ang = 2.0 * np.pi * ((n[:, None] * k[None, :]) % _DIM) / _DIM
    c = np.cos(ang)
    s = np.sin(ang)
    s[:, 0] = (-1.0) ** n
    return c.astype(np.float32), s.astype(np.float32)


_C_TAB, _S_TAB = _trig_tables()


def _fwd_kernel(x_ref, c_ref, s_ref, cm_ref, sm_ref):
    x = x_ref[...]
    cm_ref[...] = jax.lax.dot(x, c_ref[...], precision=_PREC,
                              preferred_element_type=jnp.float32)
    sm_ref[...] = jax.lax.dot(x, s_ref[...], precision=_PREC,
                              preferred_element_type=jnp.float32)


def _select_kernel(cm_ref, sm_ref, cmw_ref, smw_ref):
    cm = cm_ref[...]
    sm = sm_ref[...]
    p = cm * cm + sm * sm
    cols = jax.lax.broadcasted_iota(jnp.int32, p.shape, 1)
    pm = jnp.where(cols == 0, -1.0, p)
    q = sm[:, 0:1] * sm[:, 0:1]

    def body(_, lohi):
        lo, hi = lohi
        mid = lo + (hi - lo) // 2
        t = jax.lax.bitcast_convert_type(mid, jnp.float32)
        cnt = (2 * jnp.sum((pm > t).astype(jnp.int32), axis=1, keepdims=True)
               + (q > t).astype(jnp.int32))
        le = cnt <= _CAP
        return jnp.where(le, lo, mid + 1), jnp.where(le, mid, hi)

    lo0 = jnp.zeros((p.shape[0], 1), jnp.int32)
    hi0 = jnp.full((p.shape[0], 1), jnp.int32(0x7F800000))
    lo, hi = jax.lax.fori_loop(0, 31, body, (lo0, hi0))
    tau = jax.lax.bitcast_convert_type(hi, jnp.float32)

    full = pm > tau
    w = 2.0 * full.astype(jnp.float32)
    nyq_gt = q > tau
    used = (2 * jnp.sum(full.astype(jnp.int32), axis=1, keepdims=True)
            + nyq_gt.astype(jnp.int32))
    spare = used < _CAP
    eq = pm == tau
    nyq_eq = q == tau
    eq_idx = jnp.where(eq, cols, 2 * _DIM)
    min_pair = jnp.min(eq_idx, axis=1, keepdims=True)
    min_k = jnp.minimum(min_pair, jnp.where(nyq_eq, _HALF, 2 * _DIM))
    w = w + (spare & eq & (cols == min_k)).astype(jnp.float32)
    w_nyq = nyq_gt.astype(jnp.float32) + (
        spare & nyq_eq & (min_k == _HALF)).astype(jnp.float32)

    wc = jnp.where(cols == 0, 1.0, w)
    ws = jnp.where(cols == 0, w_nyq, w)
    cmw_ref[...] = cm * wc
    smw_ref[...] = sm * ws


def _inv_kernel(cmw_ref, smw_ref, c_ref, s_ref, o_ref):
    dims = (((1,), (1,)), ((), ()))
    acc = jax.lax.dot_general(cmw_ref[...], c_ref[...], dims, precision=_PREC,
                              preferred_element_type=jnp.float32)
    acc += jax.lax.dot_general(smw_ref[...], s_ref[...], dims, precision=_PREC,
                               preferred_element_type=jnp.float32)
    o_ref[...] = acc * (1.0 / _DIM)


@functools.partial(jax.jit)
def kernel(gradient):
    x = gradient.astype(jnp.float32)
    c_tab = jnp.asarray(_C_TAB)
    s_tab = jnp.asarray(_S_TAB)

    fwd = pl.pallas_call(
        _fwd_kernel,
        grid=(_ROWS // _BM, _HALF // _BNF),
        in_specs=[
            pl.BlockSpec((_BM, _DIM), lambda i, j: (i, 0)),
            pl.BlockSpec((_DIM, _BNF), lambda i, j: (0, j)),
            pl.BlockSpec((_DIM, _BNF), lambda i, j: (0, j)),
        ],
        out_specs=[
            pl.BlockSpec((_BM, _BNF), lambda i, j: (i, j)),
            pl.BlockSpec((_BM, _BNF), lambda i, j: (i, j)),
        ],
        out_shape=[
            jax.ShapeDtypeStruct((_ROWS, _HALF), jnp.float32),
            jax.ShapeDtypeStruct((_ROWS, _HALF), jnp.float32),
        ],
    )
    cm, sm = fwd(x, c_tab, s_tab)

    sel = pl.pallas_call(
        _select_kernel,
        grid=(_ROWS // _BM,),
        in_specs=[
            pl.BlockSpec((_BM, _HALF), lambda i: (i, 0)),
            pl.BlockSpec((_BM, _HALF), lambda i: (i, 0)),
        ],
        out_specs=[
            pl.BlockSpec((_BM, _HALF), lambda i: (i, 0)),
            pl.BlockSpec((_BM, _HALF), lambda i: (i, 0)),
        ],
        out_shape=[
            jax.ShapeDtypeStruct((_ROWS, _HALF), jnp.float32),
            jax.ShapeDtypeStruct((_ROWS, _HALF), jnp.float32),
        ],
    )
    cmw, smw = sel(cm, sm)

    inv = pl.pallas_call(
        _inv_kernel,
        grid=(_ROWS // _BM, _DIM // _BNI),
        in_specs=[
            pl.BlockSpec((_BM, _HALF), lambda i, j: (i, 0)),
            pl.BlockSpec((_BM, _HALF), lambda i, j: (i, 0)),
            pl.BlockSpec((_BNI, _HALF), lambda i, j: (j, 0)),
            pl.BlockSpec((_BNI, _HALF), lambda i, j: (j, 0)),
        ],
        out_specs=pl.BlockSpec((_BM, _BNI), lambda i, j: (i, j)),
        out_shape=jax.ShapeDtypeStruct((_ROWS, _DIM), jnp.float32),
    )
    return inv(cmw, smw, c_tab, s_tab)


# speedup vs baseline: 121.7065x; 1.6925x over previous
import functools

import jax
import jax.numpy as jnp
import numpy as np
from jax.experimental import pallas as pl

_ROWS = 2048
_DIM = 4096
_HALF = _DIM // 2
_BW = 256
_CAP = _BW - 1

_BM = 256
_BNF = 256
_BNI = 512

def _trig_tables():
    n = np.arange(_DIM)
    k = np.arange(_HALF)
    ang = 2.0 * np.pi * ((n[:, None] * k[None, :]) % _DIM) / _DIM
    c = np.cos(ang)
    s = np.sin(ang)
    s[:, 0] = (-1.0) ** n
    return c.astype(np.float32), s.astype(np.float32)


def _split_bf16(a):
    hi = a.astype(jnp.bfloat16)
    lo = (a - hi.astype(np.float32)).astype(jnp.bfloat16)
    return hi, lo


_C_TAB, _S_TAB = _trig_tables()
_C_HI, _C_LO = _split_bf16(_C_TAB)
_S_HI, _S_LO = _split_bf16(_S_TAB)


def _split3(x):
    xh = x.astype(jnp.bfloat16)
    xl = (x - xh.astype(jnp.float32)).astype(jnp.bfloat16)
    return xh, xl


def _dot3(xh, xl, t_hi, t_lo, dims=None):
    if dims is None:
        f = lambda a, b: jax.lax.dot(a, b, preferred_element_type=jnp.float32)
    else:
        f = lambda a, b: jax.lax.dot_general(
            a, b, dims, preferred_element_type=jnp.float32)
    return f(xh, t_hi) + (f(xh, t_lo) + f(xl, t_hi))


def _fwd_kernel(x_ref, ch_ref, cl_ref, sh_ref, sl_ref, cm_ref, sm_ref):
    xh, xl = _split3(x_ref[...])
    cm_ref[...] = _dot3(xh, xl, ch_ref[...], cl_ref[...])
    sm_ref[...] = _dot3(xh, xl, sh_ref[...], sl_ref[...])


def _select_kernel(cm_ref, sm_ref, cmw_ref, smw_ref):
    cm = cm_ref[...]
    sm = sm_ref[...]
    p = cm * cm + sm * sm
    cols = jax.lax.broadcasted_iota(jnp.int32, p.shape, 1)
    pm = jnp.where(cols == 0, -1.0, p)
    q = sm[:, 0:1] * sm[:, 0:1]

    def body(_, lohi):
        lo, hi = lohi
        mid = lo + (hi - lo) // 2
        t = jax.lax.bitcast_convert_type(mid, jnp.float32)
        cnt = (2 * jnp.sum((pm > t).astype(jnp.int32), axis=1, keepdims=True)
               + (q > t).astype(jnp.int32))
        le = cnt <= _CAP
        return jnp.where(le, lo, mid + 1), jnp.where(le, mid, hi)

    lo0 = jnp.zeros((p.shape[0], 1), jnp.int32)
    hi0 = jnp.full((p.shape[0], 1), jnp.int32(0x7F800000))
    lo, hi = jax.lax.fori_loop(0, 31, body, (lo0, hi0))
    tau = jax.lax.bitcast_convert_type(hi, jnp.float32)

    full = pm > tau
    w = 2.0 * full.astype(jnp.float32)
    nyq_gt = q > tau
    used = (2 * jnp.sum(full.astype(jnp.int32), axis=1, keepdims=True)
            + nyq_gt.astype(jnp.int32))
    spare = used < _CAP
    eq = pm == tau
    nyq_eq = q == tau
    eq_idx = jnp.where(eq, cols, 2 * _DIM)
    min_pair = jnp.min(eq_idx, axis=1, keepdims=True)
    min_k = jnp.minimum(min_pair, jnp.where(nyq_eq, _HALF, 2 * _DIM))
    w = w + (spare & eq & (cols == min_k)).astype(jnp.float32)
    w_nyq = nyq_gt.astype(jnp.float32) + (
        spare & nyq_eq & (min_k == _HALF)).astype(jnp.float32)

    wc = jnp.where(cols == 0, 1.0, w)
    ws = jnp.where(cols == 0, w_nyq, w)
    cmw_ref[...] = cm * wc
    smw_ref[...] = sm * ws


def _inv_kernel(cmw_ref, smw_ref, ch_ref, cl_ref, sh_ref, sl_ref, o_ref):
    dims = (((1,), (1,)), ((), ()))
    ah, al = _split3(cmw_ref[...])
    bh, bl = _split3(smw_ref[...])
    acc = _dot3(ah, al, ch_ref[...], cl_ref[...], dims)
    acc += _dot3(bh, bl, sh_ref[...], sl_ref[...], dims)
    o_ref[...] = acc * (1.0 / _DIM)


@functools.partial(jax.jit)
def kernel(gradient):
    x = gradient.astype(jnp.float32)
    c_hi, c_lo = jnp.asarray(_C_HI), jnp.asarray(_C_LO)
    s_hi, s_lo = jnp.asarray(_S_HI), jnp.asarray(_S_LO)

    fwd = pl.pallas_call(
        _fwd_kernel,
        grid=(_ROWS // _BM, _HALF // _BNF),
        in_specs=[
            pl.BlockSpec((_BM, _DIM), lambda i, j: (i, 0)),
            pl.BlockSpec((_DIM, _BNF), lambda i, j: (0, j)),
            pl.BlockSpec((_DIM, _BNF), lambda i, j: (0, j)),
            pl.BlockSpec((_DIM, _BNF), lambda i, j: (0, j)),
            pl.BlockSpec((_DIM, _BNF), lambda i, j: (0, j)),
        ],
        out_specs=[
            pl.BlockSpec((_BM, _BNF), lambda i, j: (i, j)),
            pl.BlockSpec((_BM, _BNF), lambda i, j: (i, j)),
        ],
        out_shape=[
            jax.ShapeDtypeStruct((_ROWS, _HALF), jnp.float32),
            jax.ShapeDtypeStruct((_ROWS, _HALF), jnp.float32),
        ],
    )
    cm, sm = fwd(x, c_hi, c_lo, s_hi, s_lo)

    sel = pl.pallas_call(
        _select_kernel,
        grid=(_ROWS // _BM,),
        in_specs=[
            pl.BlockSpec((_BM, _HALF), lambda i: (i, 0)),
            pl.BlockSpec((_BM, _HALF), lambda i: (i, 0)),
        ],
        out_specs=[
            pl.BlockSpec((_BM, _HALF), lambda i: (i, 0)),
            pl.BlockSpec((_BM, _HALF), lambda i: (i, 0)),
        ],
        out_shape=[
            jax.ShapeDtypeStruct((_ROWS, _HALF), jnp.float32),
            jax.ShapeDtypeStruct((_ROWS, _HALF), jnp.float32),
        ],
    )
    cmw, smw = sel(cm, sm)

    inv = pl.pallas_call(
        _inv_kernel,
        grid=(_ROWS // _BM, _DIM // _BNI),
        in_specs=[
            pl.BlockSpec((_BM, _HALF), lambda i, j: (i, 0)),
            pl.BlockSpec((_BM, _HALF), lambda i, j: (i, 0)),
            pl.BlockSpec((_BNI, _HALF), lambda i, j: (j, 0)),
            pl.BlockSpec((_BNI, _HALF), lambda i, j: (j, 0)),
            pl.BlockSpec((_BNI, _HALF), lambda i, j: (j, 0)),
            pl.BlockSpec((_BNI, _HALF), lambda i, j: (j, 0)),
        ],
        out_specs=pl.BlockSpec((_BM, _BNI), lambda i, j: (i, j)),
        out_shape=jax.ShapeDtypeStruct((_ROWS, _DIM), jnp.float32),
    )
    return inv(cmw, smw, c_hi, c_lo, s_hi, s_lo)


# speedup vs baseline: 136.0241x; 1.1176x over previous
import functools

import jax
import jax.numpy as jnp
import numpy as np
from jax.experimental import pallas as pl

_ROWS = 2048
_DIM = 4096
_HALF = _DIM // 2
_BW = 256
_CAP = _BW - 1

_BM = 256
_BNF = 256
_BNI = 512

def _trig_tables():
    n = np.arange(_DIM)
    k = np.arange(_HALF)
    ang = 2.0 * np.pi * ((n[:, None] * k[None, :]) % _DIM) / _DIM
    c = np.cos(ang)
    s = np.sin(ang)
    s[:, 0] = (-1.0) ** n
    return c.astype(np.float32), s.astype(np.float32)


def _split_bf16(a):
    hi = a.astype(jnp.bfloat16)
    lo = (a - hi.astype(np.float32)).astype(jnp.bfloat16)
    return hi, lo


_C_TAB, _S_TAB = _trig_tables()
_C_HI, _C_LO = _split_bf16(_C_TAB)
_S_HI, _S_LO = _split_bf16(_S_TAB)


def _split3(x):
    xh = x.astype(jnp.bfloat16)
    xl = (x - xh.astype(jnp.float32)).astype(jnp.bfloat16)
    return xh, xl


def _dot3(xh, xl, t_hi, t_lo, dims=None):
    if dims is None:
        f = lambda a, b: jax.lax.dot(a, b, preferred_element_type=jnp.float32)
    else:
        f = lambda a, b: jax.lax.dot_general(
            a, b, dims, preferred_element_type=jnp.float32)
    return f(xh, t_hi) + (f(xh, t_lo) + f(xl, t_hi))


def _fwd_kernel(x_ref, ch_ref, cl_ref, sh_ref, sl_ref, cm_ref, sm_ref):
    xh, xl = _split3(x_ref[...])
    cm_ref[...] = _dot3(xh, xl, ch_ref[...], cl_ref[...])
    sm_ref[...] = _dot3(xh, xl, sh_ref[...], sl_ref[...])


def _select_kernel(cm_ref, sm_ref, cmw_ref, smw_ref):
    cm = cm_ref[...]
    sm = sm_ref[...]
    p = cm * cm + sm * sm
    cols = jax.lax.broadcasted_iota(jnp.int32, p.shape, 1)
    pm = jnp.where(cols == 0, -1.0, p)
    q = sm[:, 0:1] * sm[:, 0:1]

    def body(_, lohi):
        lo, hi = lohi
        mid = lo + (hi - lo) // 2
        t = jax.lax.bitcast_convert_type(mid, jnp.float32)
        cnt = (2 * jnp.sum((pm > t).astype(jnp.int32), axis=1, keepdims=True)
               + (q > t).astype(jnp.int32))
        le = cnt <= _CAP
        return jnp.where(le, lo, mid + 1), jnp.where(le, mid, hi)

    lo0 = jnp.zeros((p.shape[0], 1), jnp.int32)
    hi0 = jnp.full((p.shape[0], 1), jnp.int32(0x7F800000))
    lo, hi = jax.lax.fori_loop(0, 31, body, (lo0, hi0))
    tau = jax.lax.bitcast_convert_type(hi, jnp.float32)

    full = pm > tau
    w = 2.0 * full.astype(jnp.float32)
    nyq_gt = q > tau
    used = (2 * jnp.sum(full.astype(jnp.int32), axis=1, keepdims=True)
            + nyq_gt.astype(jnp.int32))
    spare = used < _CAP
    eq = pm == tau
    nyq_eq = q == tau
    eq_idx = jnp.where(eq, cols, 2 * _DIM)
    min_pair = jnp.min(eq_idx, axis=1, keepdims=True)
    min_k = jnp.minimum(min_pair, jnp.where(nyq_eq, _HALF, 2 * _DIM))
    w = w + (spare & eq & (cols == min_k)).astype(jnp.float32)
    w_nyq = nyq_gt.astype(jnp.float32) + (
        spare & nyq_eq & (min_k == _HALF)).astype(jnp.float32)

    wc = jnp.where(cols == 0, 1.0, w)
    ws = jnp.where(cols == 0, w_nyq, w)
    cmw_ref[...] = cm * wc
    smw_ref[...] = sm * ws


def _inv_kernel(cmw_ref, smw_ref, ch_ref, cl_ref, sh_ref, sl_ref, o_ref):
    dims = (((1,), (1,)), ((), ()))
    ah = cmw_ref[...].astype(jnp.bfloat16)
    bh = smw_ref[...].astype(jnp.bfloat16)
    acc = jax.lax.dot_general(ah, ch_ref[...], dims,
                              preferred_element_type=jnp.float32)
    acc += jax.lax.dot_general(bh, sh_ref[...], dims,
                               preferred_element_type=jnp.float32)
    o_ref[...] = acc * (1.0 / _DIM)


@functools.partial(jax.jit)
def kernel(gradient):
    x = gradient.astype(jnp.float32)
    c_hi, c_lo = jnp.asarray(_C_HI), jnp.asarray(_C_LO)
    s_hi, s_lo = jnp.asarray(_S_HI), jnp.asarray(_S_LO)

    fwd = pl.pallas_call(
        _fwd_kernel,
        grid=(_ROWS // _BM, _HALF // _BNF),
        in_specs=[
            pl.BlockSpec((_BM, _DIM), lambda i, j: (i, 0)),
            pl.BlockSpec((_DIM, _BNF), lambda i, j: (0, j)),
            pl.BlockSpec((_DIM, _BNF), lambda i, j: (0, j)),
            pl.BlockSpec((_DIM, _BNF), lambda i, j: (0, j)),
            pl.BlockSpec((_DIM, _BNF), lambda i, j: (0, j)),
        ],
        out_specs=[
            pl.BlockSpec((_BM, _BNF), lambda i, j: (i, j)),
            pl.BlockSpec((_BM, _BNF), lambda i, j: (i, j)),
        ],
        out_shape=[
            jax.ShapeDtypeStruct((_ROWS, _HALF), jnp.float32),
            jax.ShapeDtypeStruct((_ROWS, _HALF), jnp.float32),
        ],
    )
    cm, sm = fwd(x, c_hi, c_lo, s_hi, s_lo)

    sel = pl.pallas_call(
        _select_kernel,
        grid=(_ROWS // _BM,),
        in_specs=[
            pl.BlockSpec((_BM, _HALF), lambda i: (i, 0)),
            pl.BlockSpec((_BM, _HALF), lambda i: (i, 0)),
        ],
        out_specs=[
            pl.BlockSpec((_BM, _HALF), lambda i: (i, 0)),
            pl.BlockSpec((_BM, _HALF), lambda i: (i, 0)),
        ],
        out_shape=[
            jax.ShapeDtypeStruct((_ROWS, _HALF), jnp.float32),
            jax.ShapeDtypeStruct((_ROWS, _HALF), jnp.float32),
        ],
    )
    cmw, smw = sel(cm, sm)

    inv = pl.pallas_call(
        _inv_kernel,
        grid=(_ROWS // _BM, _DIM // _BNI),
        in_specs=[
            pl.BlockSpec((_BM, _HALF), lambda i, j: (i, 0)),
            pl.BlockSpec((_BM, _HALF), lambda i, j: (i, 0)),
            pl.BlockSpec((_BNI, _HALF), lambda i, j: (j, 0)),
            pl.BlockSpec((_BNI, _HALF), lambda i, j: (j, 0)),
            pl.BlockSpec((_BNI, _HALF), lambda i, j: (j, 0)),
            pl.BlockSpec((_BNI, _HALF), lambda i, j: (j, 0)),
        ],
        out_specs=pl.BlockSpec((_BM, _BNI), lambda i, j: (i, j)),
        out_shape=jax.ShapeDtypeStruct((_ROWS, _DIM), jnp.float32),
    )
    return inv(cmw, smw, c_hi, c_lo, s_hi, s_lo)
